# Initial kernel scaffold; baseline (speedup 1.0000x reference)
import jax
import jax.numpy as jnp
from jax.experimental import pallas as pl


def kernel(target_feat, target_candidate, tp_W1, tp_b1, tp_g, tp_B, tp_W2, tp_b2, tm_W1, tm_b1, tm_g, tm_B, tm_W2, tm_b2, me_W1, me_b1, me_g, me_B, me_W2, me_b2, ts_W1, ts_b1, ts_g, ts_B, ts_W2, ts_b2):
    raise NotImplementedError("write your pallas kernel here")


# speedup vs baseline: 1.5789x; 1.5789x over previous
import jax
import jax.numpy as jnp
from jax.experimental import pallas as pl
from jax.experimental.pallas import tpu as pltpu

M = 50
HORIZON = 30
D = 64
H = 64
N = 50000
BLK = 7168
NB = 7
NPAD = NB * BLK
MSEL = 64


def _ln_relu_cols(hT, gc, Bc):
    mu = jnp.mean(hT, axis=0, keepdims=True)
    dd = hT - mu
    var = jnp.mean(dd * dd, axis=0, keepdims=True)
    hn = dd / jnp.sqrt(var + 1e-5) * gc + Bc
    return jnp.maximum(hn, 0.0)


def _tnt_body(featT_ref, cx2_ref, cy2_ref,
              tp_W1T_ref, tp_b1_ref, tp_g_ref, tp_B_ref, tp_w2r_ref, tp_b2_ref,
              tm_W1T_ref, tm_b1_ref, tm_g_ref, tm_B_ref, tm_W2T_ref, tm_b2_ref,
              me_W1T_ref, me_b1_ref, me_g_ref, me_B_ref, me_W2T_ref, me_b2_ref,
              ts_W1T_ref, ts_b1_ref, ts_g_ref, ts_B_ref, ts_w2r_ref, ts_b2_ref,
              trajsT_out_ref, score_out_ref, logits_s):
    i = pl.program_id(0)
    featT = featT_ref[...]

    @pl.when(i < NB)
    def _stage1():
        cxr = cx2_ref[pl.ds(i, 1), :]
        cyr = cy2_ref[pl.ds(i, 1), :]
        xbT = jnp.concatenate(
            [jnp.broadcast_to(featT, (D, BLK)), cxr, cyr], axis=0)
        hT = jnp.dot(tp_W1T_ref[...], xbT,
                     preferred_element_type=jnp.float32) + tp_b1_ref[...]
        hr = _ln_relu_cols(hT, tp_g_ref[...], tp_B_ref[...])
        lg = jnp.dot(tp_w2r_ref[...], hr,
                     preferred_element_type=jnp.float32) + tp_b2_ref[...]
        flat = i * BLK + jax.lax.broadcasted_iota(jnp.int32, (1, BLK), 1)
        logits_s[pl.ds(i, 1), :] = jnp.where(flat < N, lg, -jnp.inf)

    @pl.when(i == NB)
    def _stage2():
        L0 = logits_s[...]
        cx2 = cx2_ref[...]
        cy2 = cy2_ref[...]
        flat2 = (jax.lax.broadcasted_iota(jnp.int32, (NB, BLK), 0) * BLK
                 + jax.lax.broadcasted_iota(jnp.int32, (NB, BLK), 1))
        neg = jnp.float32(-jnp.inf)
        BIG = jnp.int32(2 ** 30)
        lane64 = jax.lax.broadcasted_iota(jnp.int32, (1, MSEL), 1)

        def sel_body(j, carry):
            L, sxT, syT = carry
            m = jnp.max(L)
            pick = jnp.min(jnp.where(L == m, flat2, BIG))
            hit = flat2 == pick
            x = jnp.sum(jnp.where(hit, cx2, 0.0))
            y = jnp.sum(jnp.where(hit, cy2, 0.0))
            L = jnp.where(hit, neg, L)
            sxT = jnp.where(lane64 == j, x, sxT)
            syT = jnp.where(lane64 == j, y, syT)
            return L, sxT, syT

        z = jnp.zeros((1, MSEL), jnp.float32)
        _, sxT, syT = jax.lax.fori_loop(0, M, sel_body, (L0, z, z))

        featB = jnp.broadcast_to(featT, (D, MSEL))

        xselT = jnp.concatenate([featB, sxT, syT], axis=0)
        h2 = jnp.dot(tm_W1T_ref[...], xselT,
                     preferred_element_type=jnp.float32) + tm_b1_ref[...]
        hr2 = _ln_relu_cols(h2, tm_g_ref[...], tm_B_ref[...])
        offT = jnp.dot(tm_W2T_ref[...], hr2,
                       preferred_element_type=jnp.float32) + tm_b2_ref[...]
        locT = jnp.concatenate([sxT, syT], axis=0) + offT

        xinT = jnp.concatenate([featB, locT], axis=0)
        h3 = jnp.dot(me_W1T_ref[...], xinT,
                     preferred_element_type=jnp.float32) + me_b1_ref[...]
        hr3 = _ln_relu_cols(h3, me_g_ref[...], me_B_ref[...])
        trajsT = jnp.dot(me_W2T_ref[...], hr3,
                         preferred_element_type=jnp.float32) + me_b2_ref[...]

        xsT = jnp.concatenate([featB, trajsT], axis=0)
        h4 = jnp.dot(ts_W1T_ref[...], xsT,
                     preferred_element_type=jnp.float32) + ts_b1_ref[...]
        hr4 = _ln_relu_cols(h4, ts_g_ref[...], ts_B_ref[...])
        slog = jnp.dot(ts_w2r_ref[...], hr4,
                       preferred_element_type=jnp.float32) + ts_b2_ref[...]
        validc = lane64 < M
        slog = jnp.where(validc, slog, neg)
        sm = jnp.max(slog)
        e = jnp.where(validc, jnp.exp(slog - sm), 0.0)
        score = e / jnp.sum(e)

        trajsT_out_ref[...] = trajsT[:, :M]
        score_out_ref[...] = score[:, :M]


def kernel(target_feat, target_candidate, tp_W1, tp_b1, tp_g, tp_B, tp_W2, tp_b2,
           tm_W1, tm_b1, tm_g, tm_B, tm_W2, tm_b2,
           me_W1, me_b1, me_g, me_B, me_W2, me_b2,
           ts_W1, ts_b1, ts_g, ts_B, ts_W2, ts_b2):
    c = jnp.pad(target_candidate, ((0, NPAD - N), (0, 0)))
    cx2 = c[:, 0].reshape(NB, BLK)
    cy2 = c[:, 1].reshape(NB, BLK)

    def col(v):
        return v.reshape(-1, 1)

    full = lambda i: (0, 0)
    fspec = lambda a: pl.BlockSpec(a.shape, full)

    args = (
        target_feat.T, cx2, cy2,
        tp_W1.T, col(tp_b1), col(tp_g), col(tp_B), tp_W2.T, tp_b2.reshape(1, 1),
        tm_W1.T, col(tm_b1), col(tm_g), col(tm_B), tm_W2.T, col(tm_b2),
        me_W1.T, col(me_b1), col(me_g), col(me_B), me_W2.T, col(me_b2),
        ts_W1.T, col(ts_b1), col(ts_g), col(ts_B), ts_W2.T, ts_b2.reshape(1, 1),
    )
    trajsT, score = pl.pallas_call(
        _tnt_body,
        grid=(NB + 1,),
        in_specs=[fspec(a) for a in args],
        out_specs=(
            pl.BlockSpec((HORIZON * 2, M), full),
            pl.BlockSpec((1, M), full),
        ),
        out_shape=(
            jax.ShapeDtypeStruct((HORIZON * 2, M), jnp.float32),
            jax.ShapeDtypeStruct((1, M), jnp.float32),
        ),
        scratch_shapes=[pltpu.VMEM((NB, BLK), jnp.float32)],
    )(*args)
    return trajsT.T, score.reshape(M)


# speedup vs baseline: 2.1592x; 1.3676x over previous
import jax
import jax.numpy as jnp
from jax.experimental import pallas as pl

M = 50
HORIZON = 30
D = 64
H = 64
N = 50000
BLK = 7168
NB = 7
MSEL = 64


def _ln_relu_cols(hT, gc, Bc):
    mu = jnp.mean(hT, axis=0, keepdims=True)
    dd = hT - mu
    var = jnp.mean(dd * dd, axis=0, keepdims=True)
    hn = dd / jnp.sqrt(var + 1e-5) * gc + Bc
    return jnp.maximum(hn, 0.0)


def _tnt_body(cxy_ref, W1_ref, W2_ref, V_ref, trajsT_out_ref, score_out_ref):
    V = V_ref[...]
    featT = V[:, 0:1]
    featB = jnp.broadcast_to(featT, (D, BLK))

    tp_W1T = W1_ref[:, 0:66]
    tp_w2r = W2_ref[0:1, :]
    tp_b1c, tp_g, tp_B = V[:, 1:2], V[:, 2:3], V[:, 3:4]
    tp_b2 = V[0:1, 15:16]

    rows = []
    for j in range(NB):
        cxr = cxy_ref[j:j + 1, :]
        cyr = cxy_ref[NB + j:NB + j + 1, :]
        xbT = jnp.concatenate([featB, cxr, cyr], axis=0)
        hT = jnp.dot(tp_W1T, xbT,
                     preferred_element_type=jnp.float32) + tp_b1c
        hr = _ln_relu_cols(hT, tp_g, tp_B)
        lg = jnp.dot(tp_w2r, hr,
                     preferred_element_type=jnp.float32) + tp_b2
        if (j + 1) * BLK > N:
            lane = jax.lax.broadcasted_iota(jnp.int32, (1, BLK), 1)
            lg = jnp.where(j * BLK + lane < N, lg, -jnp.inf)
        rows.append(lg)
    L0 = jnp.concatenate(rows, axis=0)

    cx2 = cxy_ref[0:NB, :]
    cy2 = cxy_ref[NB:2 * NB, :]
    flat2 = (jax.lax.broadcasted_iota(jnp.int32, (NB, BLK), 0) * BLK
             + jax.lax.broadcasted_iota(jnp.int32, (NB, BLK), 1))
    neg = jnp.float32(-jnp.inf)
    BIG = jnp.int32(2 ** 30)
    lane64 = jax.lax.broadcasted_iota(jnp.int32, (1, MSEL), 1)

    def sel_body(j, carry):
        L, sxT, syT = carry
        m = jnp.max(L)
        pick = jnp.min(jnp.where(L == m, flat2, BIG))
        hit = flat2 == pick
        x = jnp.sum(jnp.where(hit, cx2, 0.0))
        y = jnp.sum(jnp.where(hit, cy2, 0.0))
        L = jnp.where(hit, neg, L)
        sxT = jnp.where(lane64 == j, x, sxT)
        syT = jnp.where(lane64 == j, y, syT)
        return L, sxT, syT

    z = jnp.zeros((1, MSEL), jnp.float32)
    _, sxT, syT = jax.lax.fori_loop(0, M, sel_body, (L0, z, z))

    featB64 = jnp.broadcast_to(featT, (D, MSEL))

    xselT = jnp.concatenate([featB64, sxT, syT], axis=0)
    h2 = jnp.dot(W1_ref[:, 128:194], xselT,
                 preferred_element_type=jnp.float32) + V[:, 4:5]
    hr2 = _ln_relu_cols(h2, V[:, 5:6], V[:, 6:7])
    offT = jnp.dot(W2_ref[1:3, :], hr2,
                   preferred_element_type=jnp.float32) + V[0:2, 13:14]
    locT = jnp.concatenate([sxT, syT], axis=0) + offT

    xinT = jnp.concatenate([featB64, locT], axis=0)
    h3 = jnp.dot(W1_ref[:, 256:322], xinT,
                 preferred_element_type=jnp.float32) + V[:, 7:8]
    hr3 = _ln_relu_cols(h3, V[:, 8:9], V[:, 9:10])
    trajsT = jnp.dot(W2_ref[3:63, :], hr3,
                     preferred_element_type=jnp.float32) + V[0:60, 14:15]

    xsT = jnp.concatenate([featB64, trajsT], axis=0)
    h4 = jnp.dot(W1_ref[:, 384:508], xsT,
                 preferred_element_type=jnp.float32) + V[:, 10:11]
    hr4 = _ln_relu_cols(h4, V[:, 11:12], V[:, 12:13])
    slog = jnp.dot(W2_ref[63:64, :], hr4,
                   preferred_element_type=jnp.float32) + V[1:2, 15:16]
    validc = lane64 < M
    slog = jnp.where(validc, slog, neg)
    sm = jnp.max(slog)
    e = jnp.where(validc, jnp.exp(slog - sm), 0.0)
    score = e / jnp.sum(e)

    trajsT_out_ref[...] = trajsT[:, :M]
    score_out_ref[...] = score[:, :M]


def kernel(target_feat, target_candidate, tp_W1, tp_b1, tp_g, tp_B, tp_W2, tp_b2,
           tm_W1, tm_b1, tm_g, tm_B, tm_W2, tm_b2,
           me_W1, me_b1, me_g, me_B, me_W2, me_b2,
           ts_W1, ts_b1, ts_g, ts_B, ts_W2, ts_b2):
    c = jnp.pad(target_candidate, ((0, NB * BLK - N), (0, 0)))
    cxy = jnp.concatenate(
        [c[:, 0].reshape(NB, BLK), c[:, 1].reshape(NB, BLK)], axis=0)

    z1 = jnp.zeros((D, 128), jnp.float32)
    W1pack = jnp.concatenate([
        jnp.pad(tp_W1.T, ((0, 0), (0, 62))),
        jnp.pad(tm_W1.T, ((0, 0), (0, 62))),
        jnp.pad(me_W1.T, ((0, 0), (0, 62))),
        jnp.pad(ts_W1.T, ((0, 0), (0, 4))),
    ], axis=1)
    del z1

    W2pack = jnp.concatenate([tp_W2.T, tm_W2.T, me_W2.T, ts_W2.T], axis=0)

    def colp(v):
        v = v.reshape(-1, 1)
        return jnp.pad(v, ((0, D - v.shape[0]), (0, 0)))

    V = jnp.concatenate([
        colp(target_feat), colp(tp_b1), colp(tp_g), colp(tp_B),
        colp(tm_b1), colp(tm_g), colp(tm_B),
        colp(me_b1), colp(me_g), colp(me_B),
        colp(ts_b1), colp(ts_g), colp(ts_B),
        colp(tm_b2), colp(me_b2),
        colp(jnp.concatenate([tp_b2, ts_b2])),
    ], axis=1)

    full = lambda i: (0, 0)
    args = (cxy, W1pack, W2pack, V)
    trajsT, score = pl.pallas_call(
        _tnt_body,
        grid=(1,),
        in_specs=[pl.BlockSpec(a.shape, full) for a in args],
        out_specs=(
            pl.BlockSpec((HORIZON * 2, M), full),
            pl.BlockSpec((1, M), full),
        ),
        out_shape=(
            jax.ShapeDtypeStruct((HORIZON * 2, M), jnp.float32),
            jax.ShapeDtypeStruct((1, M), jnp.float32),
        ),
    )(*args)
    return trajsT.T, score.reshape(M)


# speedup vs baseline: 2.4788x; 1.1480x over previous
import jax
import jax.numpy as jnp
from jax.experimental import pallas as pl

M = 50
HORIZON = 30
D = 64
H = 64
N = 50000
BLK = 7168
NB = 7
MSEL = 64


def _ln_relu_cols(hT, gc, Bc):
    mu = jnp.mean(hT, axis=0, keepdims=True)
    dd = hT - mu
    var = jnp.mean(dd * dd, axis=0, keepdims=True)
    hn = dd / jnp.sqrt(var + 1e-5) * gc + Bc
    return jnp.maximum(hn, 0.0)


def _tnt_body(cxy_ref, W1_ref, W2_ref, V_ref, trajsT_out_ref, score_out_ref):
    V = V_ref[...]
    featT = V[:, 0:1]
    featB = jnp.broadcast_to(featT, (D, BLK))

    tp_W1T = W1_ref[:, 0:66]
    tp_w2r = W2_ref[0:1, :]
    tp_b1c, tp_g, tp_B = V[:, 1:2], V[:, 2:3], V[:, 3:4]
    tp_b2 = V[0:1, 15:16]

    rows = []
    for j in range(NB):
        cxr = cxy_ref[j:j + 1, :]
        cyr = cxy_ref[NB + j:NB + j + 1, :]
        xbT = jnp.concatenate([featB, cxr, cyr], axis=0)
        hT = jnp.dot(tp_W1T, xbT,
                     preferred_element_type=jnp.float32) + tp_b1c
        hr = _ln_relu_cols(hT, tp_g, tp_B)
        lg = jnp.dot(tp_w2r, hr,
                     preferred_element_type=jnp.float32) + tp_b2
        if (j + 1) * BLK > N:
            lane = jax.lax.broadcasted_iota(jnp.int32, (1, BLK), 1)
            lg = jnp.where(j * BLK + lane < N, lg, -jnp.inf)
        rows.append(lg)
    L0 = jnp.concatenate(rows, axis=0)

    cx2 = cxy_ref[0:NB, :]
    cy2 = cxy_ref[NB:2 * NB, :]
    flat2 = (jax.lax.broadcasted_iota(jnp.int32, (NB, BLK), 0) * BLK
             + jax.lax.broadcasted_iota(jnp.int32, (NB, BLK), 1))
    neg = jnp.float32(-jnp.inf)
    BIG = jnp.int32(2 ** 30)
    lane64 = jax.lax.broadcasted_iota(jnp.int32, (1, MSEL), 1)

    L = L0
    xs, ys = [], []
    for _ in range(M):
        mv = jnp.max(L, axis=(0, 1), keepdims=True)
        pickv = jnp.min(jnp.where(L == mv, flat2, BIG),
                        axis=(0, 1), keepdims=True)
        hit = flat2 == pickv
        xs.append(jnp.sum(jnp.where(hit, cx2, 0.0),
                          axis=(0, 1), keepdims=True))
        ys.append(jnp.sum(jnp.where(hit, cy2, 0.0),
                          axis=(0, 1), keepdims=True))
        L = jnp.where(hit, neg, L)
    pad14 = jnp.zeros((1, MSEL - M), jnp.float32)
    sxT = jnp.concatenate(xs + [pad14], axis=1)
    syT = jnp.concatenate(ys + [pad14], axis=1)

    featB64 = jnp.broadcast_to(featT, (D, MSEL))

    xselT = jnp.concatenate([featB64, sxT, syT], axis=0)
    h2 = jnp.dot(W1_ref[:, 128:194], xselT,
                 preferred_element_type=jnp.float32) + V[:, 4:5]
    hr2 = _ln_relu_cols(h2, V[:, 5:6], V[:, 6:7])
    offT = jnp.dot(W2_ref[1:3, :], hr2,
                   preferred_element_type=jnp.float32) + V[0:2, 13:14]
    locT = jnp.concatenate([sxT, syT], axis=0) + offT

    xinT = jnp.concatenate([featB64, locT], axis=0)
    h3 = jnp.dot(W1_ref[:, 256:322], xinT,
                 preferred_element_type=jnp.float32) + V[:, 7:8]
    hr3 = _ln_relu_cols(h3, V[:, 8:9], V[:, 9:10])
    trajsT = jnp.dot(W2_ref[3:63, :], hr3,
                     preferred_element_type=jnp.float32) + V[0:60, 14:15]

    xsT = jnp.concatenate([featB64, trajsT], axis=0)
    h4 = jnp.dot(W1_ref[:, 384:508], xsT,
                 preferred_element_type=jnp.float32) + V[:, 10:11]
    hr4 = _ln_relu_cols(h4, V[:, 11:12], V[:, 12:13])
    slog = jnp.dot(W2_ref[63:64, :], hr4,
                   preferred_element_type=jnp.float32) + V[1:2, 15:16]
    validc = lane64 < M
    slog = jnp.where(validc, slog, neg)
    sm = jnp.max(slog)
    e = jnp.where(validc, jnp.exp(slog - sm), 0.0)
    score = e / jnp.sum(e)

    trajsT_out_ref[...] = trajsT[:, :M]
    score_out_ref[...] = score[:, :M]


def kernel(target_feat, target_candidate, tp_W1, tp_b1, tp_g, tp_B, tp_W2, tp_b2,
           tm_W1, tm_b1, tm_g, tm_B, tm_W2, tm_b2,
           me_W1, me_b1, me_g, me_B, me_W2, me_b2,
           ts_W1, ts_b1, ts_g, ts_B, ts_W2, ts_b2):
    c = jnp.pad(target_candidate, ((0, NB * BLK - N), (0, 0)))
    cxy = jnp.concatenate(
        [c[:, 0].reshape(NB, BLK), c[:, 1].reshape(NB, BLK)], axis=0)

    z1 = jnp.zeros((D, 128), jnp.float32)
    W1pack = jnp.concatenate([
        jnp.pad(tp_W1.T, ((0, 0), (0, 62))),
        jnp.pad(tm_W1.T, ((0, 0), (0, 62))),
        jnp.pad(me_W1.T, ((0, 0), (0, 62))),
        jnp.pad(ts_W1.T, ((0, 0), (0, 4))),
    ], axis=1)
    del z1

    W2pack = jnp.concatenate([tp_W2.T, tm_W2.T, me_W2.T, ts_W2.T], axis=0)

    def colp(v):
        v = v.reshape(-1, 1)
        return jnp.pad(v, ((0, D - v.shape[0]), (0, 0)))

    V = jnp.concatenate([
        colp(target_feat), colp(tp_b1), colp(tp_g), colp(tp_B),
        colp(tm_b1), colp(tm_g), colp(tm_B),
        colp(me_b1), colp(me_g), colp(me_B),
        colp(ts_b1), colp(ts_g), colp(ts_B),
        colp(tm_b2), colp(me_b2),
        colp(jnp.concatenate([tp_b2, ts_b2])),
    ], axis=1)

    full = lambda i: (0, 0)
    args = (cxy, W1pack, W2pack, V)
    trajsT, score = pl.pallas_call(
        _tnt_body,
        grid=(1,),
        in_specs=[pl.BlockSpec(a.shape, full) for a in args],
        out_specs=(
            pl.BlockSpec((HORIZON * 2, M), full),
            pl.BlockSpec((1, M), full),
        ),
        out_shape=(
            jax.ShapeDtypeStruct((HORIZON * 2, M), jnp.float32),
            jax.ShapeDtypeStruct((1, M), jnp.float32),
        ),
    )(*args)
    return trajsT.T, score.reshape(M)
